# baseline (speedup 1.0000x reference)
import functools

import jax
import jax.numpy as jnp
from jax import lax
from jax.experimental import pallas as pl
from jax.experimental.pallas import tpu as pltpu
from jax.experimental.pallas import tpu_sc as plsc

_DIM = 300
_SUP = 8
_CNT = (64, 64, 72)
_COL = (0, 64, 128)


_PB1 = 208


@functools.partial(jax.jit, static_argnames=("b0", "b1"))
def _gather(table, idx_flat, b0, b1):
    info = plsc.get_sparse_core_info()
    nc, ns = info.num_cores, info.num_subcores
    nw = nc * ns
    rows_per_w = b0 // nw
    n_sup = rows_per_w // _SUP
    mesh = plsc.VectorSubcoreMesh(core_axis_name="c", subcore_axis_name="s")

    @functools.partial(
        pl.kernel,
        mesh=mesh,
        out_type=jax.ShapeDtypeStruct((b0, b1, _DIM), jnp.float32),
        scratch_types=[
            pltpu.VMEM((_SUP * _PB1,), jnp.int32),
            pltpu.VMEM((1, _CNT[0], _DIM), jnp.float32),
            pltpu.VMEM((1, _CNT[1], _DIM), jnp.float32),
            pltpu.VMEM((1, _CNT[2], _DIM), jnp.float32),
            pltpu.SemaphoreType.DMA,
            pltpu.SemaphoreType.DMA,
            pltpu.SemaphoreType.DMA,
            pltpu.SemaphoreType.DMA,
            pltpu.SemaphoreType.DMA,
            pltpu.SemaphoreType.DMA,
        ],
    )
    def k(table_hbm, idx_hbm, out_hbm, idx_v,
          rows0, rows1, rows2, g0, g1, g2, w0, w1, w2):
        wid = lax.axis_index("s") * nc + lax.axis_index("c")
        r0 = wid * rows_per_w
        rows = (rows0, rows1, rows2)
        sem_g = (g0, g1, g2)
        sem_w = (w0, w1, w2)

        def issue_gathers(jl, b):
            dst = rows[b].at[0]

            def issue16(base, lanes):
                vec = idx_v[pl.ds(jl * _PB1 + _COL[b] + base, 16)]
                for l in lanes:
                    pltpu.async_copy(
                        table_hbm.at[pl.ds(vec[l], 1)],
                        dst.at[pl.ds(base + l, 1)],
                        sem_g[b],
                    )

            def group(g, carry):
                issue16(g * 16, range(16))
                return carry

            lax.fori_loop(0, _CNT[b] // 16, group, 0)
            if _CNT[b] % 16:
                base = _CNT[b] - _CNT[b] % 16
                issue16(base, range(_CNT[b] % 16))

        def drain_gathers(b):
            pltpu.make_async_copy(
                table_hbm.at[pl.ds(0, _CNT[b])], rows[b].at[0], sem_g[b]
            ).wait()

        def write_out(s, b):
            pltpu.async_copy(
                rows[b],
                out_hbm.at[pl.ds(s, 1), pl.ds(_COL[b], _CNT[b])],
                sem_w[b],
            )

        def wait_write(b):
            pltpu.make_async_copy(
                rows[b], out_hbm.at[pl.ds(0, 1), pl.ds(_COL[b], _CNT[b])],
                sem_w[b],
            ).wait()

        def super_body(sup, carry):
            pltpu.sync_copy(
                idx_hbm.at[pl.ds((r0 + sup * _SUP) * _PB1, _SUP * _PB1)],
                idx_v,
            )

            def row_body(j, carry2):
                gj = sup * _SUP + j
                s = r0 + gj
                nonfirst = gj > 0
                pl.when(nonfirst)(lambda: wait_write(0))
                issue_gathers(j, 0)
                pl.when(nonfirst)(lambda: drain_gathers(2))
                pl.when(nonfirst)(lambda: write_out(s - 1, 2))
                pl.when(nonfirst)(lambda: wait_write(1))
                issue_gathers(j, 1)
                drain_gathers(0)
                write_out(s, 0)
                pl.when(nonfirst)(lambda: wait_write(2))
                issue_gathers(j, 2)
                drain_gathers(1)
                write_out(s, 1)
                return carry2

            lax.fori_loop(0, _SUP, row_body, 0)
            return carry

        lax.fori_loop(0, n_sup, super_body, 0)
        drain_gathers(2)
        write_out(r0 + rows_per_w - 1, 2)
        wait_write(0)
        wait_write(1)
        wait_write(2)

    return k(table, idx_flat)


def kernel(table, idxes):
    b0, b1 = idxes.shape
    idx_flat = jnp.pad(
        idxes.astype(jnp.int32), ((0, 0), (0, _PB1 - b1))
    ).reshape(b0 * _PB1)
    return _gather(table, idx_flat, b0, b1)


# speedup vs baseline: 1.0786x; 1.0786x over previous
import functools

import jax
import jax.numpy as jnp
from jax import lax
from jax.experimental import pallas as pl
from jax.experimental.pallas import tpu as pltpu
from jax.experimental.pallas import tpu_sc as plsc

_DIM = 300
_CHUNK = 64
_NBUF = 4


@functools.partial(jax.jit, static_argnames=("n_rows",))
def _gather(table, idx_flat, n_rows):
    info = plsc.get_sparse_core_info()
    nc, ns = info.num_cores, info.num_subcores
    nw = nc * ns
    chunks_per_w = n_rows // (_CHUNK * nw)
    idx_per_w = n_rows // nw
    n_quad = chunks_per_w // _NBUF
    mesh = plsc.VectorSubcoreMesh(core_axis_name="c", subcore_axis_name="s")

    @functools.partial(
        pl.kernel,
        mesh=mesh,
        out_type=jax.ShapeDtypeStruct((n_rows, _DIM), jnp.float32),
        scratch_types=[
            pltpu.VMEM((idx_per_w,), jnp.int32),
            pltpu.VMEM((_CHUNK, _DIM), jnp.float32),
            pltpu.VMEM((_CHUNK, _DIM), jnp.float32),
            pltpu.VMEM((_CHUNK, _DIM), jnp.float32),
            pltpu.VMEM((_CHUNK, _DIM), jnp.float32),
            pltpu.SemaphoreType.DMA,
            pltpu.SemaphoreType.DMA,
            pltpu.SemaphoreType.DMA,
            pltpu.SemaphoreType.DMA,
            pltpu.SemaphoreType.DMA,
            pltpu.SemaphoreType.DMA,
            pltpu.SemaphoreType.DMA,
            pltpu.SemaphoreType.DMA,
        ],
    )
    def k(table_hbm, idx_hbm, out_hbm, idx_v,
          rows0, rows1, rows2, rows3, g0, g1, g2, g3, w0, w1, w2, w3):
        wid = lax.axis_index("s") * nc + lax.axis_index("c")
        c0 = wid * chunks_per_w
        rows = (rows0, rows1, rows2, rows3)
        sem_g = (g0, g1, g2, g3)
        sem_w = (w0, w1, w2, w3)

        pltpu.sync_copy(idx_hbm.at[pl.ds(wid * idx_per_w, idx_per_w)], idx_v)

        def issue_gathers(c, b):
            def group(g, carry):
                vec = idx_v[pl.ds(c * _CHUNK + g * 16, 16)]
                for l in range(16):
                    pltpu.async_copy(
                        table_hbm.at[pl.ds(vec[l], 1)],
                        rows[b].at[pl.ds(g * 16 + l, 1)],
                        sem_g[b],
                    )
                return carry

            lax.fori_loop(0, _CHUNK // 16, group, 0)

        def drain_gathers(b):
            pltpu.make_async_copy(
                table_hbm.at[pl.ds(0, _CHUNK)], rows[b], sem_g[b]
            ).wait()

        def write_out(c, b):
            base = (c0 + c) * _CHUNK
            pltpu.async_copy(rows[b], out_hbm.at[pl.ds(base, _CHUNK)], sem_w[b])

        def wait_write(b):
            pltpu.make_async_copy(
                rows[b], out_hbm.at[pl.ds(0, _CHUNK)], sem_w[b]
            ).wait()

        def quad(q, carry):
            nonfirst = q > 0
            for b in range(_NBUF):
                c = q * _NBUF + b
                pl.when(nonfirst)(lambda b=b: wait_write(b))
                issue_gathers(c, b)
                pb = (b - 1) % _NBUF
                if b == 0:
                    pl.when(nonfirst)(lambda: drain_gathers(_NBUF - 1))
                    pl.when(nonfirst)(lambda c=c: write_out(c - 1, _NBUF - 1))
                else:
                    drain_gathers(pb)
                    write_out(c - 1, pb)
            return carry

        lax.fori_loop(0, n_quad, quad, 0)
        drain_gathers(_NBUF - 1)
        write_out(chunks_per_w - 1, _NBUF - 1)
        for b in range(_NBUF):
            wait_write(b)

    return k(table, idx_flat)


def kernel(table, idxes):
    b0, b1 = idxes.shape
    n_rows = b0 * b1
    idx_flat = idxes.reshape(n_rows).astype(jnp.int32)
    out = _gather(table, idx_flat, n_rows)
    return out.reshape(b0, b1, _DIM)
